# Initial kernel scaffold; baseline (speedup 1.0000x reference)
import jax
import jax.numpy as jnp
from jax.experimental import pallas as pl


def kernel(student_id, exercise_id, q_mask, student_emb, diff_emb, disc_emb, knowledge_emb):
    raise NotImplementedError("write your pallas kernel here")


# speedup vs baseline: 1.6551x; 1.6551x over previous
import functools

import jax
import jax.numpy as jnp
from jax import lax
from jax.experimental import pallas as pl
from jax.experimental.pallas import tpu as pltpu
from jax.experimental.pallas import tpu_sc as plsc

BATCH = 16384
DIM = 128
CHUNK = 128


@functools.cache
def _gather_kernel():
    info = plsc.get_sparse_core_info()
    nc, ns = info.num_cores, info.num_subcores
    nw = nc * ns
    b_per_w = BATCH // nw
    n_chunks = b_per_w // CHUNK
    mesh = plsc.VectorSubcoreMesh(core_axis_name="c", subcore_axis_name="s")

    @functools.partial(
        pl.kernel,
        mesh=mesh,
        out_type=(
            jax.ShapeDtypeStruct((BATCH, DIM), jnp.float32),
            jax.ShapeDtypeStruct((BATCH, DIM), jnp.float32),
            jax.ShapeDtypeStruct((BATCH,), jnp.float32),
        ),
        scratch_types=[
            pltpu.VMEM((b_per_w,), jnp.int32),
            pltpu.VMEM((b_per_w,), jnp.int32),
            pltpu.VMEM((b_per_w, DIM), jnp.float32),
            pltpu.VMEM((b_per_w,), jnp.float32),
            pltpu.SemaphoreType.DMA,
        ],
    )
    def k(sid_hbm, eid_hbm, semb_hbm, demb_hbm, disc_hbm,
          sout_hbm, dout_hbm, discout_hbm,
          sidx_v, eidx_v, rows_v, disc_v, gsem):
        wid = lax.axis_index("s") * nc + lax.axis_index("c")
        base = wid * b_per_w
        pltpu.sync_copy(sid_hbm.at[pl.ds(base, b_per_w)], sidx_v)
        pltpu.sync_copy(eid_hbm.at[pl.ds(base, b_per_w)], eidx_v)

        handles = []
        for j in range(n_chunks):
            sl = pl.ds(j * CHUNK, CHUNK)
            handles.append(
                pltpu.async_copy(semb_hbm.at[sidx_v.at[sl]], rows_v.at[sl], gsem))
        for h in handles:
            h.wait()
        pltpu.sync_copy(rows_v, sout_hbm.at[pl.ds(base, b_per_w)])

        handles = []
        for j in range(n_chunks):
            sl = pl.ds(j * CHUNK, CHUNK)
            handles.append(
                pltpu.async_copy(demb_hbm.at[eidx_v.at[sl]], rows_v.at[sl], gsem))
        for h in handles:
            h.wait()
        pltpu.sync_copy(rows_v, dout_hbm.at[pl.ds(base, b_per_w)])

        handles = []
        for j in range(n_chunks):
            sl = pl.ds(j * CHUNK, CHUNK)
            handles.append(
                pltpu.async_copy(disc_hbm.at[eidx_v.at[sl]], disc_v.at[sl], gsem))
        for h in handles:
            h.wait()
        pltpu.sync_copy(disc_v, discout_hbm.at[pl.ds(base, b_per_w)])

    return k


def kernel(student_id, exercise_id, q_mask, student_emb, diff_emb, disc_emb, knowledge_emb):
    del q_mask
    disc_flat = disc_emb.reshape(-1)
    student_ts, diff_ts, disc_out = _gather_kernel()(
        student_id, exercise_id, student_emb, diff_emb, disc_flat)
    return (student_ts, diff_ts, disc_out.reshape(-1, 1), knowledge_emb)


# speedup vs baseline: 1.7230x; 1.0410x over previous
import functools

import jax
import jax.numpy as jnp
from jax import lax
from jax.experimental import pallas as pl
from jax.experimental.pallas import tpu as pltpu
from jax.experimental.pallas import tpu_sc as plsc

BATCH = 16384
DIM = 128
CHUNK = 128


@functools.cache
def _gather_kernel():
    info = plsc.get_sparse_core_info()
    nc, ns = info.num_cores, info.num_subcores
    nw = nc * ns
    b_per_w = BATCH // nw
    n_chunks = b_per_w // CHUNK
    mesh = plsc.VectorSubcoreMesh(core_axis_name="c", subcore_axis_name="s")

    @functools.partial(
        pl.kernel,
        mesh=mesh,
        out_type=(
            jax.ShapeDtypeStruct((BATCH, DIM), jnp.float32),
            jax.ShapeDtypeStruct((BATCH, DIM), jnp.float32),
            jax.ShapeDtypeStruct((BATCH,), jnp.float32),
        ),
        scratch_types=[
            pltpu.VMEM((b_per_w,), jnp.int32),
            pltpu.VMEM((b_per_w,), jnp.int32),
            pltpu.VMEM((6, CHUNK, DIM), jnp.float32),
            pltpu.VMEM((b_per_w,), jnp.float32),
        ] + [pltpu.SemaphoreType.DMA] * 13,
    )
    def k(sid_hbm, eid_hbm, semb_hbm, demb_hbm, disc_hbm,
          sout_hbm, dout_hbm, discout_hbm,
          sidx_v, eidx_v, rows_v, disc_v, *sems):
        gsems, wsems, dsem = sems[0:6], sems[6:12], sems[12]
        wid = lax.axis_index("s") * nc + lax.axis_index("c")
        base = wid * b_per_w
        pltpu.sync_copy(sid_hbm.at[pl.ds(base, b_per_w)], sidx_v)
        pltpu.sync_copy(eid_hbm.at[pl.ds(base, b_per_w)], eidx_v)

        disc_handles = [
            pltpu.async_copy(
                disc_hbm.at[eidx_v.at[pl.ds(j * CHUNK, CHUNK)]],
                disc_v.at[pl.ds(j * CHUNK, CHUNK)], dsem)
            for j in range(n_chunks)
        ]

        ntask = 2 * n_chunks
        nslot = 6

        def gather_task(t, slot):
            j = t % n_chunks
            sl = pl.ds(j * CHUNK, CHUNK)
            src = semb_hbm.at[sidx_v.at[sl]] if t < n_chunks \
                else demb_hbm.at[eidx_v.at[sl]]
            return pltpu.async_copy(src, rows_v.at[slot], gsems[slot])

        def write_task(t, slot):
            j = t % n_chunks
            dst = (sout_hbm if t < n_chunks else dout_hbm
                   ).at[pl.ds(base + j * CHUNK, CHUNK)]
            return pltpu.async_copy(rows_v.at[slot], dst, wsems[slot])

        g = [None] * ntask
        w = [None] * ntask
        for t in range(nslot):
            g[t] = gather_task(t, t)
        for t in range(ntask):
            g[t].wait()
            w[t] = write_task(t, t % nslot)
            nt = t + nslot
            if nt < ntask:
                w[t].wait()
                g[nt] = gather_task(nt, nt % nslot)
        for t in range(ntask - nslot, ntask):
            w[t].wait()

        for h in disc_handles:
            h.wait()
        pltpu.sync_copy(disc_v, discout_hbm.at[pl.ds(base, b_per_w)])

    return k


def kernel(student_id, exercise_id, q_mask, student_emb, diff_emb, disc_emb, knowledge_emb):
    del q_mask
    disc_flat = disc_emb.reshape(-1)
    student_ts, diff_ts, disc_out = _gather_kernel()(
        student_id, exercise_id, student_emb, diff_emb, disc_flat)
    return (student_ts, diff_ts, disc_out.reshape(-1, 1), knowledge_emb)


# speedup vs baseline: 1.7773x; 1.0315x over previous
import functools

import jax
import jax.numpy as jnp
from jax import lax
from jax.experimental import pallas as pl
from jax.experimental.pallas import tpu as pltpu
from jax.experimental.pallas import tpu_sc as plsc

BATCH = 16384
DIM = 128
CHUNK = 128


@functools.cache
def _gather_kernel():
    info = plsc.get_sparse_core_info()
    nc, ns = info.num_cores, info.num_subcores
    nw = nc * ns
    b_per_w = BATCH // nw
    n_chunks = b_per_w // CHUNK
    mesh = plsc.VectorSubcoreMesh(core_axis_name="c", subcore_axis_name="s")

    @functools.partial(
        pl.kernel,
        mesh=mesh,
        out_type=(
            jax.ShapeDtypeStruct((BATCH, DIM), jnp.float32),
            jax.ShapeDtypeStruct((BATCH, DIM), jnp.float32),
            jax.ShapeDtypeStruct((BATCH,), jnp.float32),
        ),
        scratch_types=[
            pltpu.VMEM((b_per_w,), jnp.int32),
            pltpu.VMEM((b_per_w,), jnp.int32),
            pltpu.VMEM((7, CHUNK, DIM), jnp.float32),
            pltpu.VMEM((b_per_w,), jnp.float32),
        ] + [pltpu.SemaphoreType.DMA] * 17,
    )
    def k(sid_hbm, eid_hbm, semb_hbm, demb_hbm, disc_hbm,
          sout_hbm, dout_hbm, discout_hbm,
          sidx_v, eidx_v, rows_v, disc_v, *sems):
        gsems, wsems, dsem, isem_s, isem_e = (
            sems[0:7], sems[7:14], sems[14], sems[15], sems[16])
        wid = lax.axis_index("s") * nc + lax.axis_index("c")
        base = wid * b_per_w
        h_sidx = pltpu.async_copy(sid_hbm.at[pl.ds(base, b_per_w)], sidx_v, isem_s)
        h_eidx = pltpu.async_copy(eid_hbm.at[pl.ds(base, b_per_w)], eidx_v, isem_e)

        ntask = 2 * n_chunks
        nslot = 7

        def gather_task(t, slot):
            j = t % n_chunks
            sl = pl.ds(j * CHUNK, CHUNK)
            src = semb_hbm.at[sidx_v.at[sl]] if t < n_chunks \
                else demb_hbm.at[eidx_v.at[sl]]
            return pltpu.async_copy(src, rows_v.at[slot], gsems[slot])

        def write_task(t, slot):
            j = t % n_chunks
            dst = (sout_hbm if t < n_chunks else dout_hbm
                   ).at[pl.ds(base + j * CHUNK, CHUNK)]
            return pltpu.async_copy(rows_v.at[slot], dst, wsems[slot])

        g = [None] * ntask
        w = [None] * ntask
        h_sidx.wait()
        for t in range(n_chunks):
            g[t] = gather_task(t, t)
        h_eidx.wait()
        disc_handles = [
            pltpu.async_copy(
                disc_hbm.at[eidx_v.at[pl.ds(j * CHUNK, CHUNK)]],
                disc_v.at[pl.ds(j * CHUNK, CHUNK)], dsem)
            for j in range(n_chunks)
        ]
        for t in range(n_chunks, nslot):
            g[t] = gather_task(t, t)
        for t in range(ntask):
            g[t].wait()
            w[t] = write_task(t, t % nslot)
            nt = t + nslot
            if nt < ntask:
                w[t].wait()
                g[nt] = gather_task(nt, nt % nslot)
        for t in range(ntask - nslot, ntask):
            w[t].wait()

        for h in disc_handles:
            h.wait()
        pltpu.sync_copy(disc_v, discout_hbm.at[pl.ds(base, b_per_w)])

    return k


def kernel(student_id, exercise_id, q_mask, student_emb, diff_emb, disc_emb, knowledge_emb):
    del q_mask
    student_ts, diff_ts, disc_ts = _gather_kernel()(
        student_id, exercise_id, student_emb, diff_emb, disc_emb.reshape(-1))
    return (student_ts, diff_ts, disc_ts.reshape(-1, 1), knowledge_emb)
